# Initial kernel scaffold; baseline (speedup 1.0000x reference)
import jax
import jax.numpy as jnp
from jax.experimental import pallas as pl


def kernel(x, expert_bias, gate_weight):
    raise NotImplementedError("write your pallas kernel here")


# speedup vs baseline: 1.6012x; 1.6012x over previous
import jax
import jax.numpy as jnp
from jax import lax
from jax.experimental import pallas as pl
from jax.experimental.pallas import tpu as pltpu

NUM_EXPERTS = 64
TOP_K = 8
NUM_GROUPS = 8
EXPERTS_PER_GROUP = 8
NUM_LIMITED_GROUPS = 4
ROUTE_SCALE = 2.5

_NEG = float("-inf")


def _router_body(x_ref, wt_ref, b_ref, ts_ref, idx_ref, cnt_ref):
    B = x_ref.shape[0]
    logits = jnp.dot(x_ref[...], wt_ref[...], preferred_element_type=jnp.float32)
    scores = 1.0 / (1.0 + jnp.exp(-logits))
    biased = scores + b_ref[...]

    g_sums = []
    for g in range(NUM_GROUPS):
        blk = biased[:, g * EXPERTS_PER_GROUP:(g + 1) * EXPERTS_PER_GROUP]
        m1 = jnp.max(blk, axis=1, keepdims=True)
        eq = blk == m1
        c = jnp.sum(eq.astype(jnp.float32), axis=1, keepdims=True)
        m2 = jnp.max(jnp.where(eq, _NEG, blk), axis=1, keepdims=True)
        g_sums.append(jnp.where(c >= 2.0, m1 + m1, m1 + m2))
    gsum = jnp.concatenate(g_sums, axis=1)

    iota_g = lax.broadcasted_iota(jnp.int32, (B, NUM_GROUPS), 1)
    sel = jnp.zeros((B, NUM_GROUPS), jnp.float32)
    work = gsum
    for _ in range(NUM_LIMITED_GROUPS):
        m = jnp.max(work, axis=1, keepdims=True)
        gi = jnp.min(jnp.where(work == m, iota_g, NUM_GROUPS), axis=1,
                     keepdims=True)
        chosen = iota_g == gi
        sel = jnp.maximum(sel, chosen.astype(jnp.float32))
        work = jnp.where(chosen, _NEG, work)

    selx = jnp.concatenate(
        [jnp.broadcast_to(sel[:, g:g + 1], (B, EXPERTS_PER_GROUP))
         for g in range(NUM_GROUPS)], axis=1)
    sfc = jnp.where(selx > 0.0, biased, _NEG)

    iota_e = lax.broadcasted_iota(jnp.int32, (B, NUM_EXPERTS), 1)
    idx_cols, sc_cols = [], []
    hist = jnp.zeros((1, NUM_EXPERTS), jnp.float32)
    work = sfc
    for _ in range(TOP_K):
        m = jnp.max(work, axis=1, keepdims=True)
        ei = jnp.min(jnp.where(work == m, iota_e, NUM_EXPERTS), axis=1,
                     keepdims=True)
        oh = iota_e == ei
        sck = jnp.sum(jnp.where(oh, scores, 0.0), axis=1, keepdims=True)
        hist = hist + jnp.sum(oh.astype(jnp.float32), axis=0, keepdims=True)
        work = jnp.where(oh, _NEG, work)
        idx_cols.append(ei)
        sc_cols.append(sck)

    idx = jnp.concatenate(idx_cols, axis=1)
    ts = jnp.concatenate(sc_cols, axis=1)
    denom = jnp.sum(ts, axis=1, keepdims=True) + 1e-20
    ts = ts / denom * ROUTE_SCALE

    ts_ref[...] = ts
    idx_ref[...] = idx

    @pl.when(pl.program_id(0) == 0)
    def _init():
        cnt_ref[...] = jnp.zeros_like(cnt_ref)

    cnt_ref[...] = cnt_ref[...] + hist


def kernel(x, expert_bias, gate_weight):
    nt, d = x.shape
    B = 256
    wt = gate_weight.T
    bias2 = expert_bias.reshape(1, NUM_EXPERTS)
    ts, idx, cnt = pl.pallas_call(
        _router_body,
        grid=(nt // B,),
        in_specs=[
            pl.BlockSpec((B, d), lambda i: (i, 0)),
            pl.BlockSpec((d, NUM_EXPERTS), lambda i: (0, 0)),
            pl.BlockSpec((1, NUM_EXPERTS), lambda i: (0, 0)),
        ],
        out_specs=[
            pl.BlockSpec((B, TOP_K), lambda i: (i, 0)),
            pl.BlockSpec((B, TOP_K), lambda i: (i, 0)),
            pl.BlockSpec((1, NUM_EXPERTS), lambda i: (0, 0)),
        ],
        out_shape=[
            jax.ShapeDtypeStruct((nt, TOP_K), jnp.float32),
            jax.ShapeDtypeStruct((nt, TOP_K), jnp.int32),
            jax.ShapeDtypeStruct((1, NUM_EXPERTS), jnp.float32),
        ],
        compiler_params=pltpu.CompilerParams(
            dimension_semantics=("arbitrary",)),
    )(x, wt, bias2)
    return ts, idx, cnt.reshape(NUM_EXPERTS)


# speedup vs baseline: 4.8165x; 3.0081x over previous
import jax
import jax.numpy as jnp
from jax import lax
from jax.experimental import pallas as pl
from jax.experimental.pallas import tpu as pltpu

NUM_EXPERTS = 64
TOP_K = 8
NUM_GROUPS = 8
EXPERTS_PER_GROUP = 8
NUM_LIMITED_GROUPS = 4
ROUTE_SCALE = 2.5

_NEG = float("-inf")


def _router_body(x_ref, wt_ref, b_ref, ts_ref, idx_ref, cnt_ref):
    B = x_ref.shape[0]
    logits = jnp.dot(x_ref[...], wt_ref[...], preferred_element_type=jnp.float32)
    lt = logits.T
    scores = 1.0 / (1.0 + jnp.exp(-lt))
    biased = scores + b_ref[...]

    g = biased.reshape(NUM_GROUPS, EXPERTS_PER_GROUP, B)
    m1 = jnp.max(g, axis=1)
    eq = g == m1[:, None, :]
    c = jnp.sum(eq.astype(jnp.float32), axis=1)
    m2 = jnp.max(jnp.where(eq, _NEG, g), axis=1)
    gsum = jnp.where(c >= 2.0, m1 + m1, m1 + m2)

    iota_g = lax.broadcasted_iota(jnp.int32, (NUM_GROUPS, B), 0)
    sel = jnp.zeros((NUM_GROUPS, B), jnp.float32)
    work = gsum
    for _ in range(NUM_LIMITED_GROUPS):
        m = jnp.max(work, axis=0, keepdims=True)
        gi = jnp.min(jnp.where(work == m, iota_g, NUM_GROUPS), axis=0,
                     keepdims=True)
        chosen = iota_g == gi
        sel = jnp.maximum(sel, chosen.astype(jnp.float32))
        work = jnp.where(chosen, _NEG, work)

    selx = jnp.broadcast_to(sel[:, None, :],
                            (NUM_GROUPS, EXPERTS_PER_GROUP, B))
    selx = selx.reshape(NUM_EXPERTS, B)
    sfc = jnp.where(selx > 0.0, biased, _NEG)

    iota_e = lax.broadcasted_iota(jnp.int32, (NUM_EXPERTS, B), 0)
    idx_rows, sc_rows = [], []
    oh_acc = jnp.zeros((NUM_EXPERTS, B), jnp.float32)
    work = sfc
    for _ in range(TOP_K):
        m = jnp.max(work, axis=0, keepdims=True)
        ei = jnp.min(jnp.where(work == m, iota_e, NUM_EXPERTS), axis=0,
                     keepdims=True)
        oh = iota_e == ei
        sck = jnp.sum(jnp.where(oh, scores, 0.0), axis=0, keepdims=True)
        oh_acc = oh_acc + oh.astype(jnp.float32)
        work = jnp.where(oh, _NEG, work)
        idx_rows.append(ei)
        sc_rows.append(sck)

    idx = jnp.concatenate(idx_rows, axis=0)
    ts = jnp.concatenate(sc_rows, axis=0)
    denom = jnp.sum(ts, axis=0, keepdims=True) + 1e-20
    ts = ts / denom * ROUTE_SCALE
    hist = jnp.sum(oh_acc, axis=1, keepdims=True)

    ts_ref[...] = ts
    idx_ref[...] = idx

    @pl.when(pl.program_id(0) == 0)
    def _init():
        cnt_ref[...] = jnp.zeros_like(cnt_ref)

    cnt_ref[...] = cnt_ref[...] + hist


def kernel(x, expert_bias, gate_weight):
    nt, d = x.shape
    B = 256
    wt = gate_weight.T
    bias_col = expert_bias.reshape(NUM_EXPERTS, 1)
    ts_t, idx_t, cnt = pl.pallas_call(
        _router_body,
        grid=(nt // B,),
        in_specs=[
            pl.BlockSpec((B, d), lambda i: (i, 0)),
            pl.BlockSpec((d, NUM_EXPERTS), lambda i: (0, 0)),
            pl.BlockSpec((NUM_EXPERTS, 1), lambda i: (0, 0)),
        ],
        out_specs=[
            pl.BlockSpec((TOP_K, B), lambda i: (0, i)),
            pl.BlockSpec((TOP_K, B), lambda i: (0, i)),
            pl.BlockSpec((NUM_EXPERTS, 1), lambda i: (0, 0)),
        ],
        out_shape=[
            jax.ShapeDtypeStruct((TOP_K, nt), jnp.float32),
            jax.ShapeDtypeStruct((TOP_K, nt), jnp.int32),
            jax.ShapeDtypeStruct((NUM_EXPERTS, 1), jnp.float32),
        ],
        compiler_params=pltpu.CompilerParams(
            dimension_semantics=("arbitrary",)),
    )(x, wt, bias_col)
    return ts_t.T, idx_t.T, cnt.reshape(NUM_EXPERTS)


# speedup vs baseline: 5.9713x; 1.2398x over previous
import jax
import jax.numpy as jnp
from jax import lax
from jax.experimental import pallas as pl
from jax.experimental.pallas import tpu as pltpu

NUM_EXPERTS = 64
TOP_K = 8
NUM_GROUPS = 8
EXPERTS_PER_GROUP = 8
NUM_LIMITED_GROUPS = 4
ROUTE_SCALE = 2.5

_NEG = float("-inf")


def _router_body(x_ref, wt_ref, b_ref, ts_ref, idx_ref, cnt_ref):
    B = x_ref.shape[0]
    logits = jnp.dot(x_ref[...], wt_ref[...], preferred_element_type=jnp.float32)
    lt = logits.T
    scores = 1.0 / (1.0 + jnp.exp(-lt))
    biased = scores + b_ref[...]

    g = biased.reshape(NUM_GROUPS, EXPERTS_PER_GROUP, B)
    m1 = jnp.max(g, axis=1)
    eq = g == m1[:, None, :]
    c = jnp.sum(eq.astype(jnp.float32), axis=1)
    m2 = jnp.max(jnp.where(eq, _NEG, g), axis=1)
    gsum = jnp.where(c >= 2.0, m1 + m1, m1 + m2)

    iota_g = lax.broadcasted_iota(jnp.int32, (NUM_GROUPS, B), 0)
    sel = jnp.zeros((NUM_GROUPS, B), jnp.float32)
    work = gsum
    for _ in range(NUM_LIMITED_GROUPS):
        m = jnp.max(work, axis=0, keepdims=True)
        gi = jnp.min(jnp.where(work == m, iota_g, NUM_GROUPS), axis=0,
                     keepdims=True)
        chosen = iota_g == gi
        sel = jnp.maximum(sel, chosen.astype(jnp.float32))
        work = jnp.where(chosen, _NEG, work)

    selx = jnp.broadcast_to(sel[:, None, :],
                            (NUM_GROUPS, EXPERTS_PER_GROUP, B))
    selx = selx.reshape(NUM_EXPERTS, B)
    sfc = jnp.where(selx > 0.0, biased, _NEG)

    iota_e = lax.broadcasted_iota(jnp.int32, (NUM_EXPERTS, B), 0)
    idx_rows, sc_rows = [], []
    oh_acc = jnp.zeros((NUM_EXPERTS, B), jnp.float32)
    work = sfc
    for _ in range(TOP_K):
        m = jnp.max(work, axis=0, keepdims=True)
        ei = jnp.min(jnp.where(work == m, iota_e, NUM_EXPERTS), axis=0,
                     keepdims=True)
        oh = iota_e == ei
        sck = jnp.sum(jnp.where(oh, scores, 0.0), axis=0, keepdims=True)
        oh_acc = oh_acc + oh.astype(jnp.float32)
        work = jnp.where(oh, _NEG, work)
        idx_rows.append(ei)
        sc_rows.append(sck)

    idx = jnp.concatenate(idx_rows, axis=0)
    ts = jnp.concatenate(sc_rows, axis=0)
    denom = jnp.sum(ts, axis=0, keepdims=True) + 1e-20
    ts = ts / denom * ROUTE_SCALE
    hist = jnp.sum(oh_acc, axis=1, keepdims=True)

    ts_ref[...] = ts
    idx_ref[...] = idx

    @pl.when(pl.program_id(0) == 0)
    def _init():
        cnt_ref[...] = jnp.zeros_like(cnt_ref)

    cnt_ref[...] = cnt_ref[...] + hist


def kernel(x, expert_bias, gate_weight):
    nt, d = x.shape
    B = 512
    wt = gate_weight.T
    bias_col = expert_bias.reshape(NUM_EXPERTS, 1)
    ts_t, idx_t, cnt = pl.pallas_call(
        _router_body,
        grid=(nt // B,),
        in_specs=[
            pl.BlockSpec((B, d), lambda i: (i, 0)),
            pl.BlockSpec((d, NUM_EXPERTS), lambda i: (0, 0)),
            pl.BlockSpec((NUM_EXPERTS, 1), lambda i: (0, 0)),
        ],
        out_specs=[
            pl.BlockSpec((TOP_K, B), lambda i: (0, i)),
            pl.BlockSpec((TOP_K, B), lambda i: (0, i)),
            pl.BlockSpec((NUM_EXPERTS, 1), lambda i: (0, 0)),
        ],
        out_shape=[
            jax.ShapeDtypeStruct((TOP_K, nt), jnp.float32),
            jax.ShapeDtypeStruct((TOP_K, nt), jnp.int32),
            jax.ShapeDtypeStruct((NUM_EXPERTS, 1), jnp.float32),
        ],
        compiler_params=pltpu.CompilerParams(
            dimension_semantics=("arbitrary",)),
    )(x, wt, bias_col)
    return ts_t.T, idx_t.T, cnt.reshape(NUM_EXPERTS)


# speedup vs baseline: 6.6321x; 1.1107x over previous
import jax
import jax.numpy as jnp
from jax import lax
from jax.experimental import pallas as pl
from jax.experimental.pallas import tpu as pltpu

NUM_EXPERTS = 64
TOP_K = 8
NUM_GROUPS = 8
EXPERTS_PER_GROUP = 8
NUM_LIMITED_GROUPS = 4
ROUTE_SCALE = 2.5

_NEG = float("-inf")


def _router_body(x_ref, wt_ref, b_ref, ts_ref, idx_ref, cnt_ref):
    B = x_ref.shape[0]
    logits = jnp.dot(x_ref[...], wt_ref[...], preferred_element_type=jnp.float32)
    lt = logits.T
    scores = 1.0 / (1.0 + jnp.exp(-lt))
    biased = scores + b_ref[...]

    g = biased.reshape(NUM_GROUPS, EXPERTS_PER_GROUP, B)
    m1 = jnp.max(g, axis=1)
    eq = g == m1[:, None, :]
    c = jnp.sum(eq.astype(jnp.float32), axis=1)
    m2 = jnp.max(jnp.where(eq, _NEG, g), axis=1)
    gsum = jnp.where(c >= 2.0, m1 + m1, m1 + m2)

    iota_g = lax.broadcasted_iota(jnp.int32, (NUM_GROUPS, B), 0)
    sel = jnp.zeros((NUM_GROUPS, B), jnp.float32)
    work = gsum
    for _ in range(NUM_LIMITED_GROUPS):
        m = jnp.max(work, axis=0, keepdims=True)
        gi = jnp.min(jnp.where(work == m, iota_g, NUM_GROUPS), axis=0,
                     keepdims=True)
        chosen = iota_g == gi
        sel = jnp.maximum(sel, chosen.astype(jnp.float32))
        work = jnp.where(chosen, _NEG, work)

    selx = jnp.broadcast_to(sel[:, None, :],
                            (NUM_GROUPS, EXPERTS_PER_GROUP, B))
    selx = selx.reshape(NUM_EXPERTS, B)
    sfc = jnp.where(selx > 0.0, biased, _NEG)

    iota_e = lax.broadcasted_iota(jnp.int32, (NUM_EXPERTS, B), 0)
    idx_rows, sc_rows = [], []
    oh_acc = jnp.zeros((NUM_EXPERTS, B), jnp.float32)
    work = sfc
    for _ in range(TOP_K):
        m = jnp.max(work, axis=0, keepdims=True)
        ei = jnp.min(jnp.where(work == m, iota_e, NUM_EXPERTS), axis=0,
                     keepdims=True)
        oh = iota_e == ei
        sck = jnp.sum(jnp.where(oh, scores, 0.0), axis=0, keepdims=True)
        oh_acc = oh_acc + oh.astype(jnp.float32)
        work = jnp.where(oh, _NEG, work)
        idx_rows.append(ei)
        sc_rows.append(sck)

    idx = jnp.concatenate(idx_rows, axis=0)
    ts = jnp.concatenate(sc_rows, axis=0)
    denom = jnp.sum(ts, axis=0, keepdims=True) + 1e-20
    ts = ts / denom * ROUTE_SCALE
    hist = jnp.sum(oh_acc, axis=1, keepdims=True)

    ts_ref[...] = ts
    idx_ref[...] = idx

    @pl.when(pl.program_id(0) == 0)
    def _init():
        cnt_ref[...] = jnp.zeros_like(cnt_ref)

    cnt_ref[...] = cnt_ref[...] + hist


def kernel(x, expert_bias, gate_weight):
    nt, d = x.shape
    B = 1024
    wt = gate_weight.T
    bias_col = expert_bias.reshape(NUM_EXPERTS, 1)
    ts_t, idx_t, cnt = pl.pallas_call(
        _router_body,
        grid=(nt // B,),
        in_specs=[
            pl.BlockSpec((B, d), lambda i: (i, 0)),
            pl.BlockSpec((d, NUM_EXPERTS), lambda i: (0, 0)),
            pl.BlockSpec((NUM_EXPERTS, 1), lambda i: (0, 0)),
        ],
        out_specs=[
            pl.BlockSpec((TOP_K, B), lambda i: (0, i)),
            pl.BlockSpec((TOP_K, B), lambda i: (0, i)),
            pl.BlockSpec((NUM_EXPERTS, 1), lambda i: (0, 0)),
        ],
        out_shape=[
            jax.ShapeDtypeStruct((TOP_K, nt), jnp.float32),
            jax.ShapeDtypeStruct((TOP_K, nt), jnp.int32),
            jax.ShapeDtypeStruct((NUM_EXPERTS, 1), jnp.float32),
        ],
        compiler_params=pltpu.CompilerParams(
            dimension_semantics=("arbitrary",)),
    )(x, wt, bias_col)
    return ts_t.T, idx_t.T, cnt.reshape(NUM_EXPERTS)
